# Initial kernel scaffold; baseline (speedup 1.0000x reference)
import jax
import jax.numpy as jnp
from jax.experimental import pallas as pl


def kernel(x, edge_index, W1, b1, W_mu, b_mu, W_ls, b_ls, W_proj, b_proj):
    raise NotImplementedError("write your pallas kernel here")


# speedup vs baseline: 8.3538x; 8.3538x over previous
import functools

import jax
import jax.numpy as jnp
from jax import lax
from jax.experimental import pallas as pl
from jax.experimental.pallas import tpu as pltpu
from jax.experimental.pallas import tpu_sc as plsc

_N = 10000
_NP = 10240
_NE = 320000
_NEP = 327680
_NW = 32
_EPW = _NEP // _NW
_CH = 128
_NCH = _EPW // _CH
_RPT = _NP // 16
_DW = 16
_F = 128

_mesh = plsc.VectorSubcoreMesh(
    core_axis_name="c", subcore_axis_name="s", num_cores=2, num_subcores=16
)


def _deg_body(dst_hbm, ones_hbm, zeros_hbm, d0, d1, acc, ones_v, dst_v):
    c = lax.axis_index("c")
    s = lax.axis_index("s")
    wid = s * 2 + c
    row0 = s * _RPT
    pltpu.sync_copy(zeros_hbm, acc.at[pl.ds(row0, _RPT)])
    pltpu.sync_copy(ones_hbm, ones_v)
    plsc.subcore_barrier()

    def body(i, carry):
        base = wid * _EPW + i * _CH
        pltpu.sync_copy(dst_hbm.at[pl.ds(base, _CH)], dst_v)
        pltpu.sync_copy(ones_v, acc.at[dst_v], add=True)
        return carry

    lax.fori_loop(0, _NCH, body, 0)
    plsc.subcore_barrier()

    @pl.when(c == 0)
    def _():
        pltpu.sync_copy(acc.at[pl.ds(row0, _RPT)], d0.at[pl.ds(row0, _RPT)])

    @pl.when(c == 1)
    def _():
        pltpu.sync_copy(acc.at[pl.ds(row0, _RPT)], d1.at[pl.ds(row0, _RPT)])


_deg_kernel = functools.partial(
    pl.kernel,
    out_type=[
        jax.ShapeDtypeStruct((_NP, _DW), jnp.float32),
        jax.ShapeDtypeStruct((_NP, _DW), jnp.float32),
    ],
    mesh=_mesh,
    scratch_types=[
        pltpu.VMEM_SHARED((_NP, _DW), jnp.float32),
        pltpu.VMEM((_CH, _DW), jnp.float32),
        pltpu.VMEM((_CH,), jnp.int32),
    ],
)(_deg_body)


def _agg_body(xs_hbm, src_hbm, dst_hbm, zeros_hbm, p0, p1,
              acc, src_v, dst_v, rows_v, sem):
    c = lax.axis_index("c")
    s = lax.axis_index("s")
    wid = s * 2 + c
    row0 = s * _RPT
    pltpu.sync_copy(zeros_hbm, acc.at[pl.ds(row0, _RPT)])
    plsc.subcore_barrier()

    def body(i, carry):
        base = wid * _EPW + i * _CH
        pltpu.sync_copy(src_hbm.at[pl.ds(base, _CH)], src_v)
        pltpu.sync_copy(dst_hbm.at[pl.ds(base, _CH)], dst_v)
        pltpu.async_copy(xs_hbm.at[src_v], rows_v, sem).wait()
        pltpu.sync_copy(rows_v, acc.at[dst_v], add=True)
        return carry

    lax.fori_loop(0, _NCH, body, 0)
    plsc.subcore_barrier()

    @pl.when(c == 0)
    def _():
        pltpu.sync_copy(acc.at[pl.ds(row0, _RPT)], p0.at[pl.ds(row0, _RPT)])

    @pl.when(c == 1)
    def _():
        pltpu.sync_copy(acc.at[pl.ds(row0, _RPT)], p1.at[pl.ds(row0, _RPT)])


_agg_kernel = functools.partial(
    pl.kernel,
    out_type=[
        jax.ShapeDtypeStruct((_NP, _F), jnp.float32),
        jax.ShapeDtypeStruct((_NP, _F), jnp.float32),
    ],
    mesh=_mesh,
    scratch_types=[
        pltpu.VMEM_SHARED((_NP, _F), jnp.float32),
        pltpu.VMEM((_CH,), jnp.int32),
        pltpu.VMEM((_CH,), jnp.int32),
        pltpu.VMEM((_CH, _F), jnp.float32),
        pltpu.SemaphoreType.DMA,
    ],
)(_agg_body)


_BLK = 1024
_GRID = _NP // _BLK


def _scale_body(d0_ref, d1_ref, x_ref, dinv_ref, xs_ref):
    deg = d0_ref[:, 0:1] + d1_ref[:, 0:1] + 1.0
    dinv = lax.rsqrt(deg)
    dinv_ref[...] = dinv
    xs_ref[...] = dinv * x_ref[...]


def _scale_call(d0, d1, x_pad):
    return pl.pallas_call(
        _scale_body,
        grid=(_GRID,),
        in_specs=[
            pl.BlockSpec((_BLK, _DW), lambda i: (i, 0)),
            pl.BlockSpec((_BLK, _DW), lambda i: (i, 0)),
            pl.BlockSpec((_BLK, _F), lambda i: (i, 0)),
        ],
        out_specs=[
            pl.BlockSpec((_BLK, 1), lambda i: (i, 0)),
            pl.BlockSpec((_BLK, _F), lambda i: (i, 0)),
        ],
        out_shape=[
            jax.ShapeDtypeStruct((_NP, 1), jnp.float32),
            jax.ShapeDtypeStruct((_NP, _F), jnp.float32),
        ],
    )(d0, d1, x_pad)


def _hidden_body(p0_ref, p1_ref, dinv_ref, x_ref, w1_ref, b1_ref,
                 wp_ref, bp_ref, hid_ref, hs_ref):
    dv = dinv_ref[...]
    x = x_ref[...]
    agg = dv * (p0_ref[...] + p1_ref[...]) + dv * dv * x
    h = jnp.dot(agg, w1_ref[...], preferred_element_type=jnp.float32)
    h = jnp.maximum(h + b1_ref[...], 0.0)
    h = h + jnp.dot(x, wp_ref[...], preferred_element_type=jnp.float32)
    h = h + bp_ref[...]
    hid_ref[...] = h
    hs_ref[...] = dv * h


def _hidden_call(p0, p1, dinv, x_pad, W1, b1, W_proj, b_proj):
    full = lambda i: (0, 0)
    return pl.pallas_call(
        _hidden_body,
        grid=(_GRID,),
        in_specs=[
            pl.BlockSpec((_BLK, _F), lambda i: (i, 0)),
            pl.BlockSpec((_BLK, _F), lambda i: (i, 0)),
            pl.BlockSpec((_BLK, 1), lambda i: (i, 0)),
            pl.BlockSpec((_BLK, _F), lambda i: (i, 0)),
            pl.BlockSpec((_F, _F), full),
            pl.BlockSpec((1, _F), full),
            pl.BlockSpec((_F, _F), full),
            pl.BlockSpec((1, _F), full),
        ],
        out_specs=[
            pl.BlockSpec((_BLK, _F), lambda i: (i, 0)),
            pl.BlockSpec((_BLK, _F), lambda i: (i, 0)),
        ],
        out_shape=[
            jax.ShapeDtypeStruct((_NP, _F), jnp.float32),
            jax.ShapeDtypeStruct((_NP, _F), jnp.float32),
        ],
    )(p0, p1, dinv, x_pad, W1, b1, W_proj, b_proj)


def _heads_body(q0_ref, q1_ref, dinv_ref, hid_ref, wm_ref, bm_ref,
                wl_ref, bl_ref, mu_ref, ls_ref):
    dv = dinv_ref[...]
    agg = dv * (q0_ref[...] + q1_ref[...]) + dv * dv * hid_ref[...]
    mu_ref[...] = (
        jnp.dot(agg, wm_ref[...], preferred_element_type=jnp.float32)
        + bm_ref[...]
    )
    ls_ref[...] = (
        jnp.dot(agg, wl_ref[...], preferred_element_type=jnp.float32)
        + bl_ref[...]
    )


def _heads_call(q0, q1, dinv, hid, W_mu, b_mu, W_ls, b_ls):
    full = lambda i: (0, 0)
    oc = W_mu.shape[1]
    return pl.pallas_call(
        _heads_body,
        grid=(_GRID,),
        in_specs=[
            pl.BlockSpec((_BLK, _F), lambda i: (i, 0)),
            pl.BlockSpec((_BLK, _F), lambda i: (i, 0)),
            pl.BlockSpec((_BLK, 1), lambda i: (i, 0)),
            pl.BlockSpec((_BLK, _F), lambda i: (i, 0)),
            pl.BlockSpec((_F, oc), full),
            pl.BlockSpec((1, oc), full),
            pl.BlockSpec((_F, oc), full),
            pl.BlockSpec((1, oc), full),
        ],
        out_specs=[
            pl.BlockSpec((_BLK, oc), lambda i: (i, 0)),
            pl.BlockSpec((_BLK, oc), lambda i: (i, 0)),
        ],
        out_shape=[
            jax.ShapeDtypeStruct((_NP, oc), jnp.float32),
            jax.ShapeDtypeStruct((_NP, oc), jnp.float32),
        ],
    )(q0, q1, dinv, hid, W_mu, b_mu, W_ls, b_ls)


def kernel(x, edge_index, W1, b1, W_mu, b_mu, W_ls, b_ls, W_proj, b_proj):
    ei = edge_index.astype(jnp.int32)
    pad_idx = jnp.full((_NEP - _NE,), _N, jnp.int32)
    src = jnp.concatenate([ei[0], pad_idx])
    dst = jnp.concatenate([ei[1], pad_idx])
    x_pad = jnp.zeros((_NP, _F), x.dtype).at[:_N].set(x)

    ones_dw = jnp.zeros((_CH, _DW), jnp.float32).at[:, 0].set(1.0)
    zeros_dw = jnp.zeros((_RPT, _DW), jnp.float32)
    zeros_f = jnp.zeros((_RPT, _F), jnp.float32)

    d0, d1 = _deg_kernel(dst, ones_dw, zeros_dw)
    dinv, xs = _scale_call(d0, d1, x_pad)
    p0, p1 = _agg_kernel(xs, src, dst, zeros_f)
    hid, hs = _hidden_call(p0, p1, dinv, x_pad, W1, b1.reshape(1, -1),
                           W_proj, b_proj.reshape(1, -1))
    q0, q1 = _agg_kernel(hs, src, dst, zeros_f)
    mu, ls = _heads_call(q0, q1, dinv, hid, W_mu, b_mu.reshape(1, -1),
                         W_ls, b_ls.reshape(1, -1))
    return (mu[:_N], ls[:_N])


# speedup vs baseline: 15.1613x; 1.8149x over previous
import functools

import jax
import jax.numpy as jnp
from jax import lax
from jax.experimental import pallas as pl
from jax.experimental.pallas import tpu as pltpu
from jax.experimental.pallas import tpu_sc as plsc

_N = 10000
_NP = 10240
_NE = 320000
_NEP = 327680
_NW = 32
_EPW = _NEP // _NW
_CH = 128
_NCH = _EPW // _CH
_RPT = _NP // 16
_DW = 16
_F = 128

_mesh = plsc.VectorSubcoreMesh(
    core_axis_name="c", subcore_axis_name="s", num_cores=2, num_subcores=16
)


def _deg_body(dst_hbm, ones_hbm, zeros_hbm, d0, d1, acc, ones_v, dst_v):
    c = lax.axis_index("c")
    s = lax.axis_index("s")
    wid = s * 2 + c
    row0 = s * _RPT
    pltpu.sync_copy(zeros_hbm, acc.at[pl.ds(row0, _RPT)])
    pltpu.sync_copy(ones_hbm, ones_v)
    plsc.subcore_barrier()

    def body(i, carry):
        base = wid * _EPW + i * _CH
        pltpu.sync_copy(dst_hbm.at[pl.ds(base, _CH)], dst_v)
        pltpu.sync_copy(ones_v, acc.at[dst_v], add=True)
        return carry

    lax.fori_loop(0, _NCH, body, 0)
    plsc.subcore_barrier()

    @pl.when(c == 0)
    def _():
        pltpu.sync_copy(acc.at[pl.ds(row0, _RPT)], d0.at[pl.ds(row0, _RPT)])

    @pl.when(c == 1)
    def _():
        pltpu.sync_copy(acc.at[pl.ds(row0, _RPT)], d1.at[pl.ds(row0, _RPT)])


_deg_kernel = functools.partial(
    pl.kernel,
    out_type=[
        jax.ShapeDtypeStruct((_NP, _DW), jnp.float32),
        jax.ShapeDtypeStruct((_NP, _DW), jnp.float32),
    ],
    mesh=_mesh,
    scratch_types=[
        pltpu.VMEM_SHARED((_NP, _DW), jnp.float32),
        pltpu.VMEM((_CH, _DW), jnp.float32),
        pltpu.VMEM((_CH,), jnp.int32),
    ],
)(_deg_body)


_NBUF = 4
_H = 64
_EPW2 = _NEP // 16
_NCH2 = _EPW2 // _CH


def _agg_body(xs_hbm, src_hbm, dst_hbm, zeros_hbm, p0, p1,
              acc, src_all, dst_all, rows0, rows1, rows2, rows3,
              sem0, sem1, sem2, sem3):
    c = lax.axis_index("c")
    s = lax.axis_index("s")
    row0 = s * _RPT
    bufs = (rows0, rows1, rows2, rows3)
    sems = (sem0, sem1, sem2, sem3)
    pltpu.sync_copy(zeros_hbm, acc.at[pl.ds(row0, _RPT)])
    pltpu.sync_copy(src_hbm.at[c * 16 + s], src_all)
    pltpu.sync_copy(dst_hbm.at[s], dst_all)
    plsc.subcore_barrier()

    for b in range(_NBUF):
        pltpu.async_copy(xs_hbm.at[src_all.at[b]], bufs[b], sems[b])

    def outer(k, carry):
        for b in range(_NBUF):
            i = k * _NBUF + b
            pltpu.make_async_copy(xs_hbm.at[src_all.at[b]],
                                  bufs[b], sems[b]).wait()
            pltpu.sync_copy(bufs[b], acc.at[dst_all.at[i]], add=True)

            @pl.when(i + _NBUF < _NCH2)
            def _():
                pltpu.async_copy(xs_hbm.at[src_all.at[i + _NBUF]],
                                 bufs[b], sems[b])

        return carry

    lax.fori_loop(0, _NCH2 // _NBUF, outer, 0)
    plsc.subcore_barrier()

    @pl.when(c == 0)
    def _():
        pltpu.sync_copy(acc.at[pl.ds(row0, _RPT)], p0.at[pl.ds(row0, _RPT)])

    @pl.when(c == 1)
    def _():
        pltpu.sync_copy(acc.at[pl.ds(row0, _RPT)], p1.at[pl.ds(row0, _RPT)])


_agg_kernel = functools.partial(
    pl.kernel,
    out_type=[
        jax.ShapeDtypeStruct((_NP, _H), jnp.float32),
        jax.ShapeDtypeStruct((_NP, _H), jnp.float32),
    ],
    mesh=_mesh,
    scratch_types=[
        pltpu.VMEM_SHARED((_NP, _H), jnp.float32),
        pltpu.VMEM((_NCH2, _CH), jnp.int32),
        pltpu.VMEM((_NCH2, _CH), jnp.int32),
        pltpu.VMEM((_CH, _H), jnp.float32),
        pltpu.VMEM((_CH, _H), jnp.float32),
        pltpu.VMEM((_CH, _H), jnp.float32),
        pltpu.VMEM((_CH, _H), jnp.float32),
        pltpu.SemaphoreType.DMA,
        pltpu.SemaphoreType.DMA,
        pltpu.SemaphoreType.DMA,
        pltpu.SemaphoreType.DMA,
    ],
    compiler_params=pltpu.CompilerParams(use_tc_tiling_on_sc=False),
)(_agg_body)


_BLK = 1024
_GRID = _NP // _BLK


def _scale_body(d0_ref, d1_ref, x_ref, dinv_ref, xs2_ref):
    deg = d0_ref[:, 0:1] + d1_ref[:, 0:1] + 1.0
    dinv = lax.rsqrt(deg)
    dinv_ref[...] = dinv
    xs = dinv * x_ref[...]
    xs2_ref[0] = xs[:, :_H]
    xs2_ref[1] = xs[:, _H:]


def _scale_call(d0, d1, x_pad):
    return pl.pallas_call(
        _scale_body,
        grid=(_GRID,),
        in_specs=[
            pl.BlockSpec((_BLK, _DW), lambda i: (i, 0)),
            pl.BlockSpec((_BLK, _DW), lambda i: (i, 0)),
            pl.BlockSpec((_BLK, _F), lambda i: (i, 0)),
        ],
        out_specs=[
            pl.BlockSpec((_BLK, 1), lambda i: (i, 0)),
            pl.BlockSpec((2, _BLK, _H), lambda i: (0, i, 0)),
        ],
        out_shape=[
            jax.ShapeDtypeStruct((_NP, 1), jnp.float32),
            jax.ShapeDtypeStruct((2, _NP, _H), jnp.float32),
        ],
    )(d0, d1, x_pad)


def _hidden_body(p0_ref, p1_ref, dinv_ref, x_ref, w1_ref, b1_ref,
                 wp_ref, bp_ref, hid_ref, hs2_ref):
    dv = dinv_ref[...]
    x = x_ref[...]
    p = jnp.concatenate([p0_ref[...], p1_ref[...]], axis=1)
    agg = dv * p + dv * dv * x
    h = jnp.dot(agg, w1_ref[...], preferred_element_type=jnp.float32)
    h = jnp.maximum(h + b1_ref[...], 0.0)
    h = h + jnp.dot(x, wp_ref[...], preferred_element_type=jnp.float32)
    h = h + bp_ref[...]
    hid_ref[...] = h
    hs = dv * h
    hs2_ref[0] = hs[:, :_H]
    hs2_ref[1] = hs[:, _H:]


def _hidden_call(p0, p1, dinv, x_pad, W1, b1, W_proj, b_proj):
    full = lambda i: (0, 0)
    return pl.pallas_call(
        _hidden_body,
        grid=(_GRID,),
        in_specs=[
            pl.BlockSpec((_BLK, _H), lambda i: (i, 0)),
            pl.BlockSpec((_BLK, _H), lambda i: (i, 0)),
            pl.BlockSpec((_BLK, 1), lambda i: (i, 0)),
            pl.BlockSpec((_BLK, _F), lambda i: (i, 0)),
            pl.BlockSpec((_F, _F), full),
            pl.BlockSpec((1, _F), full),
            pl.BlockSpec((_F, _F), full),
            pl.BlockSpec((1, _F), full),
        ],
        out_specs=[
            pl.BlockSpec((_BLK, _F), lambda i: (i, 0)),
            pl.BlockSpec((2, _BLK, _H), lambda i: (0, i, 0)),
        ],
        out_shape=[
            jax.ShapeDtypeStruct((_NP, _F), jnp.float32),
            jax.ShapeDtypeStruct((2, _NP, _H), jnp.float32),
        ],
    )(p0, p1, dinv, x_pad, W1, b1, W_proj, b_proj)


def _heads_body(q0_ref, q1_ref, dinv_ref, hid_ref, wm_ref, bm_ref,
                wl_ref, bl_ref, mu_ref, ls_ref):
    dv = dinv_ref[...]
    q = jnp.concatenate([q0_ref[...], q1_ref[...]], axis=1)
    agg = dv * q + dv * dv * hid_ref[...]
    mu_ref[...] = (
        jnp.dot(agg, wm_ref[...], preferred_element_type=jnp.float32)
        + bm_ref[...]
    )
    ls_ref[...] = (
        jnp.dot(agg, wl_ref[...], preferred_element_type=jnp.float32)
        + bl_ref[...]
    )


def _heads_call(q0, q1, dinv, hid, W_mu, b_mu, W_ls, b_ls):
    full = lambda i: (0, 0)
    oc = W_mu.shape[1]
    return pl.pallas_call(
        _heads_body,
        grid=(_GRID,),
        in_specs=[
            pl.BlockSpec((_BLK, _H), lambda i: (i, 0)),
            pl.BlockSpec((_BLK, _H), lambda i: (i, 0)),
            pl.BlockSpec((_BLK, 1), lambda i: (i, 0)),
            pl.BlockSpec((_BLK, _F), lambda i: (i, 0)),
            pl.BlockSpec((_F, oc), full),
            pl.BlockSpec((1, oc), full),
            pl.BlockSpec((_F, oc), full),
            pl.BlockSpec((1, oc), full),
        ],
        out_specs=[
            pl.BlockSpec((_BLK, oc), lambda i: (i, 0)),
            pl.BlockSpec((_BLK, oc), lambda i: (i, 0)),
        ],
        out_shape=[
            jax.ShapeDtypeStruct((_NP, oc), jnp.float32),
            jax.ShapeDtypeStruct((_NP, oc), jnp.float32),
        ],
    )(q0, q1, dinv, hid, W_mu, b_mu, W_ls, b_ls)


def kernel(x, edge_index, W1, b1, W_mu, b_mu, W_ls, b_ls, W_proj, b_proj):
    ei = edge_index.astype(jnp.int32)
    pad_idx = jnp.full((_NEP - _NE,), _N, jnp.int32)
    src = jnp.concatenate([ei[0], pad_idx])
    dst = jnp.concatenate([ei[1], pad_idx])
    x_pad = jnp.zeros((_NP, _F), x.dtype).at[:_N].set(x)

    ones_dw = jnp.zeros((_CH, _DW), jnp.float32).at[:, 0].set(1.0)
    zeros_dw = jnp.zeros((_RPT, _DW), jnp.float32)
    zeros_h = jnp.zeros((_RPT, _H), jnp.float32)

    sbase = src.reshape(16, _NCH2, _CH)
    src_off = jnp.concatenate([sbase, sbase + _NP])
    dst4 = dst.reshape(16, _NCH2, _CH)

    d0, d1 = _deg_kernel(dst, ones_dw, zeros_dw)
    dinv, xs2 = _scale_call(d0, d1, x_pad)
    p0, p1 = _agg_kernel(xs2.reshape(2 * _NP, _H), src_off, dst4, zeros_h)
    hid, hs2 = _hidden_call(p0, p1, dinv, x_pad, W1, b1.reshape(1, -1),
                            W_proj, b_proj.reshape(1, -1))
    q0, q1 = _agg_kernel(hs2.reshape(2 * _NP, _H), src_off, dst4, zeros_h)
    mu, ls = _heads_call(q0, q1, dinv, hid, W_mu, b_mu.reshape(1, -1),
                         W_ls, b_ls.reshape(1, -1))
    return (mu[:_N], ls[:_N])


# speedup vs baseline: 31.6349x; 2.0865x over previous
import functools

import jax
import jax.numpy as jnp
from jax import lax
from jax.experimental import pallas as pl
from jax.experimental.pallas import tpu as pltpu
from jax.experimental.pallas import tpu_sc as plsc

_N = 10000
_NP = 10240
_NE = 320000
_NEP = 327680
_NW = 32
_EPW = _NEP // _NW
_CH = 128
_NCH = _EPW // _CH
_RPT = _NP // 16
_DW = 16
_F = 128

_mesh = plsc.VectorSubcoreMesh(
    core_axis_name="c", subcore_axis_name="s", num_cores=2, num_subcores=16
)


def _deg_body(dst_hbm, ones_hbm, zeros_hbm, d0, d1, acc, ones_v, dst_v):
    c = lax.axis_index("c")
    s = lax.axis_index("s")
    wid = s * 2 + c
    row0 = s * _RPT
    pltpu.sync_copy(zeros_hbm, acc.at[pl.ds(row0, _RPT)])
    pltpu.sync_copy(ones_hbm, ones_v)
    plsc.subcore_barrier()

    def body(i, carry):
        base = wid * _EPW + i * _CH
        pltpu.sync_copy(dst_hbm.at[pl.ds(base, _CH)], dst_v)
        pltpu.sync_copy(ones_v, acc.at[dst_v], add=True)
        return carry

    lax.fori_loop(0, _NCH, body, 0)
    plsc.subcore_barrier()

    @pl.when(c == 0)
    def _():
        pltpu.sync_copy(acc.at[pl.ds(row0, _RPT)], d0.at[pl.ds(row0, _RPT)])

    @pl.when(c == 1)
    def _():
        pltpu.sync_copy(acc.at[pl.ds(row0, _RPT)], d1.at[pl.ds(row0, _RPT)])


_deg_kernel = functools.partial(
    pl.kernel,
    out_type=[
        jax.ShapeDtypeStruct((_NP, _DW), jnp.float32),
        jax.ShapeDtypeStruct((_NP, _DW), jnp.float32),
    ],
    mesh=_mesh,
    scratch_types=[
        pltpu.VMEM_SHARED((_NP, _DW), jnp.float32),
        pltpu.VMEM((_CH, _DW), jnp.float32),
        pltpu.VMEM((_CH,), jnp.int32),
    ],
)(_deg_body)


_NBUF = 4
_H = 64
_EPW2 = _NEP // 16
_NCH2 = _EPW2 // _CH


def _agg_body(xs_hbm, src_hbm, dst_hbm, zeros_hbm, p0, p1,
              acc, src_all, dst_all, rows0, rows1, rows2, rows3,
              sem0, sem1, sem2, sem3):
    c = lax.axis_index("c")
    s = lax.axis_index("s")
    row0 = s * _RPT
    bufs = (rows0, rows1, rows2, rows3)
    sems = (sem0, sem1, sem2, sem3)
    pltpu.sync_copy(zeros_hbm, acc.at[pl.ds(row0, _RPT)])
    pltpu.sync_copy(src_hbm.at[c * 16 + s], src_all)
    pltpu.sync_copy(dst_hbm.at[s], dst_all)
    plsc.subcore_barrier()

    for b in range(_NBUF):
        pltpu.async_copy(xs_hbm.at[src_all.at[b]], bufs[b], sems[b])

    def outer(k, carry):
        for b in range(_NBUF):
            i = k * _NBUF + b
            pltpu.make_async_copy(xs_hbm.at[src_all.at[b]],
                                  bufs[b], sems[b]).wait()
            pltpu.sync_copy(bufs[b], acc.at[dst_all.at[i]], add=True)

            @pl.when(i + _NBUF < _NCH2)
            def _():
                pltpu.async_copy(xs_hbm.at[src_all.at[i + _NBUF]],
                                 bufs[b], sems[b])

        return carry

    lax.fori_loop(0, _NCH2 // _NBUF, outer, 0)
    plsc.subcore_barrier()

    @pl.when(c == 0)
    def _():
        pltpu.sync_copy(acc.at[pl.ds(row0, _RPT)], p0.at[pl.ds(row0, _RPT)])

    @pl.when(c == 1)
    def _():
        pltpu.sync_copy(acc.at[pl.ds(row0, _RPT)], p1.at[pl.ds(row0, _RPT)])


_agg_kernel = functools.partial(
    pl.kernel,
    out_type=[
        jax.ShapeDtypeStruct((_NP, _H), jnp.float32),
        jax.ShapeDtypeStruct((_NP, _H), jnp.float32),
    ],
    mesh=_mesh,
    scratch_types=[
        pltpu.VMEM_SHARED((_NP, _H), jnp.float32),
        pltpu.VMEM((_NCH2, _CH), jnp.int32),
        pltpu.VMEM((_NCH2, _CH), jnp.int32),
        pltpu.VMEM((_CH, _H), jnp.float32),
        pltpu.VMEM((_CH, _H), jnp.float32),
        pltpu.VMEM((_CH, _H), jnp.float32),
        pltpu.VMEM((_CH, _H), jnp.float32),
        pltpu.SemaphoreType.DMA,
        pltpu.SemaphoreType.DMA,
        pltpu.SemaphoreType.DMA,
        pltpu.SemaphoreType.DMA,
    ],
    compiler_params=pltpu.CompilerParams(use_tc_tiling_on_sc=False),
)(_agg_body)


_BLK = 1024
_GRID = _NP // _BLK


def _scale_body(d0_ref, d1_ref, x_ref, dinv_ref, xs2_ref):
    deg = d0_ref[:, 0:1] + d1_ref[:, 0:1] + 1.0
    dinv = lax.rsqrt(deg)
    dinv_ref[...] = dinv
    xs = dinv * x_ref[...]
    xs2_ref[0] = xs[:, :_H]
    xs2_ref[1] = xs[:, _H:]


def _scale_call(d0, d1, x_pad):
    return pl.pallas_call(
        _scale_body,
        grid=(_GRID,),
        in_specs=[
            pl.BlockSpec((_BLK, _DW), lambda i: (i, 0)),
            pl.BlockSpec((_BLK, _DW), lambda i: (i, 0)),
            pl.BlockSpec((_BLK, _F), lambda i: (i, 0)),
        ],
        out_specs=[
            pl.BlockSpec((_BLK, 1), lambda i: (i, 0)),
            pl.BlockSpec((2, _BLK, _H), lambda i: (0, i, 0)),
        ],
        out_shape=[
            jax.ShapeDtypeStruct((_NP, 1), jnp.float32),
            jax.ShapeDtypeStruct((2, _NP, _H), jnp.float32),
        ],
    )(d0, d1, x_pad)


def _hidden_body(p0_ref, p1_ref, dinv_ref, x_ref, w1_ref, b1_ref,
                 wp_ref, bp_ref, hid_ref, hs2_ref):
    dv = dinv_ref[...]
    x = x_ref[...]
    p = jnp.concatenate([p0_ref[...], p1_ref[...]], axis=1)
    agg = dv * p + dv * dv * x
    h = jnp.dot(agg, w1_ref[...], preferred_element_type=jnp.float32)
    h = jnp.maximum(h + b1_ref[...], 0.0)
    h = h + jnp.dot(x, wp_ref[...], preferred_element_type=jnp.float32)
    h = h + bp_ref[...]
    hid_ref[...] = h
    hs = dv * h
    hs2_ref[0] = hs[:, :_H]
    hs2_ref[1] = hs[:, _H:]


def _hidden_call(p0, p1, dinv, x_pad, W1, b1, W_proj, b_proj):
    full = lambda i: (0, 0)
    return pl.pallas_call(
        _hidden_body,
        grid=(_GRID,),
        in_specs=[
            pl.BlockSpec((_BLK, _H), lambda i: (i, 0)),
            pl.BlockSpec((_BLK, _H), lambda i: (i, 0)),
            pl.BlockSpec((_BLK, 1), lambda i: (i, 0)),
            pl.BlockSpec((_BLK, _F), lambda i: (i, 0)),
            pl.BlockSpec((_F, _F), full),
            pl.BlockSpec((1, _F), full),
            pl.BlockSpec((_F, _F), full),
            pl.BlockSpec((1, _F), full),
        ],
        out_specs=[
            pl.BlockSpec((_BLK, _F), lambda i: (i, 0)),
            pl.BlockSpec((2, _BLK, _H), lambda i: (0, i, 0)),
        ],
        out_shape=[
            jax.ShapeDtypeStruct((_NP, _F), jnp.float32),
            jax.ShapeDtypeStruct((2, _NP, _H), jnp.float32),
        ],
    )(p0, p1, dinv, x_pad, W1, b1, W_proj, b_proj)


def _heads_body(q0_ref, q1_ref, dinv_ref, hid_ref, wm_ref, bm_ref,
                wl_ref, bl_ref, mu_ref, ls_ref):
    dv = dinv_ref[...]
    q = jnp.concatenate([q0_ref[...], q1_ref[...]], axis=1)
    agg = dv * q + dv * dv * hid_ref[...]
    mu_ref[...] = (
        jnp.dot(agg, wm_ref[...], preferred_element_type=jnp.float32)
        + bm_ref[...]
    )
    ls_ref[...] = (
        jnp.dot(agg, wl_ref[...], preferred_element_type=jnp.float32)
        + bl_ref[...]
    )


def _heads_call(q0, q1, dinv, hid, W_mu, b_mu, W_ls, b_ls):
    full = lambda i: (0, 0)
    oc = W_mu.shape[1]
    return pl.pallas_call(
        _heads_body,
        grid=(_GRID,),
        in_specs=[
            pl.BlockSpec((_BLK, _H), lambda i: (i, 0)),
            pl.BlockSpec((_BLK, _H), lambda i: (i, 0)),
            pl.BlockSpec((_BLK, 1), lambda i: (i, 0)),
            pl.BlockSpec((_BLK, _F), lambda i: (i, 0)),
            pl.BlockSpec((_F, oc), full),
            pl.BlockSpec((1, oc), full),
            pl.BlockSpec((_F, oc), full),
            pl.BlockSpec((1, oc), full),
        ],
        out_specs=[
            pl.BlockSpec((_BLK, oc), lambda i: (i, 0)),
            pl.BlockSpec((_BLK, oc), lambda i: (i, 0)),
        ],
        out_shape=[
            jax.ShapeDtypeStruct((_NP, oc), jnp.float32),
            jax.ShapeDtypeStruct((_NP, oc), jnp.float32),
        ],
    )(q0, q1, dinv, hid, W_mu, b_mu, W_ls, b_ls)


def kernel(x, edge_index, W1, b1, W_mu, b_mu, W_ls, b_ls, W_proj, b_proj):
    ei = edge_index.astype(jnp.int32)
    pad_idx = _N + jnp.arange(_NEP - _NE, dtype=jnp.int32) % (_NP - _N)
    src = jnp.concatenate([ei[0], pad_idx])
    dst = jnp.concatenate([ei[1], pad_idx])
    x_pad = jnp.zeros((_NP, _F), x.dtype).at[:_N].set(x)

    ones_dw = jnp.zeros((_CH, _DW), jnp.float32).at[:, 0].set(1.0)
    zeros_dw = jnp.zeros((_RPT, _DW), jnp.float32)
    zeros_h = jnp.zeros((_RPT, _H), jnp.float32)

    sbase = src.reshape(16, _NCH2, _CH)
    src_off = jnp.concatenate([sbase, sbase + _NP])
    dst4 = dst.reshape(16, _NCH2, _CH)

    d0, d1 = _deg_kernel(dst, ones_dw, zeros_dw)
    dinv, xs2 = _scale_call(d0, d1, x_pad)
    p0, p1 = _agg_kernel(xs2.reshape(2 * _NP, _H), src_off, dst4, zeros_h)
    hid, hs2 = _hidden_call(p0, p1, dinv, x_pad, W1, b1.reshape(1, -1),
                            W_proj, b_proj.reshape(1, -1))
    q0, q1 = _agg_kernel(hs2.reshape(2 * _NP, _H), src_off, dst4, zeros_h)
    mu, ls = _heads_call(q0, q1, dinv, hid, W_mu, b_mu.reshape(1, -1),
                         W_ls, b_ls.reshape(1, -1))
    return (mu[:_N], ls[:_N])
